# baseline (speedup 1.0000x reference)
import functools

import jax
import jax.numpy as jnp
from jax import lax
from jax.experimental import pallas as pl
from jax.experimental.pallas import tpu as pltpu
from jax.experimental.pallas import tpu_sc as plsc

NC = 2
NS = 16
NW = NC * NS
L = 16


def _make_sc_kernel(B, V, D):
    OUT = D + 3 + D + D
    BPW = B // NW
    CH = 128
    NCHUNK = BPW // CH
    GROUPS = CH // L

    mesh = plsc.VectorSubcoreMesh(core_axis_name="c", subcore_axis_name="s")

    @functools.partial(
        pl.kernel,
        mesh=mesh,
        compiler_params=pltpu.CompilerParams(
            use_tc_tiling_on_sc=False, needs_layout_passes=False),
        out_type=jax.ShapeDtypeStruct((B, OUT), jnp.float32),
        scratch_types=[
            pltpu.VMEM((BPW,), jnp.int32),
            pltpu.VMEM((BPW,), jnp.float32),
            pltpu.VMEM((BPW,), jnp.float32),
            pltpu.VMEM((BPW,), jnp.float32),
            pltpu.VMEM((BPW,), jnp.int32),
            pltpu.VMEM((BPW,), jnp.int32),
            pltpu.VMEM((CH, D), jnp.float32),
            pltpu.VMEM((CH, OUT), jnp.float32),
            pltpu.SemaphoreType.DMA,
        ],
    )
    def sc_kernel(uid_hbm, age_hbm, hr_hbm, mo_hbm, occ_hbm, gen_hbm,
                  table_hbm, out_hbm,
                  idx_v, age_v, hr_v, mo_v, occ_v, gen_v, emb_v, out_v, sem):
        wid = lax.axis_index("s") * NC + lax.axis_index("c")
        base = wid * BPW

        pltpu.sync_copy(uid_hbm.at[pl.ds(base, BPW)], idx_v)
        pltpu.sync_copy(age_hbm.at[pl.ds(base, BPW)], age_v)
        pltpu.sync_copy(hr_hbm.at[pl.ds(base, BPW)], hr_v)
        pltpu.sync_copy(mo_hbm.at[pl.ds(base, BPW)], mo_v)
        pltpu.sync_copy(occ_hbm.at[pl.ds(base, BPW)], occ_v)
        pltpu.sync_copy(gen_hbm.at[pl.ds(base, BPW)], gen_v)

        iota = lax.iota(jnp.int32, L)
        zeros = jnp.zeros((L,), jnp.float32)
        ones = jnp.ones((L,), jnp.float32)

        def _zero(i, _):
            r = i // (OUT // L + 1)
            g = i % (OUT // L + 1)
            col = g * L + iota
            plsc.store_scatter(out_v, [jnp.full((L,), r, jnp.int32), col],
                               zeros, mask=col < OUT)
            return 0
        lax.fori_loop(0, CH * (OUT // L + 1), _zero, 0)

        for c in range(NCHUNK):
            pltpu.async_copy(
                table_hbm.at[idx_v.at[pl.ds(c * CH, CH)]], emb_v, sem
            ).wait()

            def _row(r, _):
                for j in range(D // L):
                    vals = emb_v[r, pl.ds(j * L, L)]
                    plsc.store_scatter(
                        out_v,
                        [jnp.full((L,), r, jnp.int32), j * L + iota],
                        vals)
                return 0
            lax.fori_loop(0, CH, _row, 0)

            def _grp(g, _):
                rows = g * L + iota
                src = c * CH + g * L
                age16 = age_v[pl.ds(src, L)]
                hr16 = hr_v[pl.ds(src, L)]
                mo16 = mo_v[pl.ds(src, L)]
                occ16 = occ_v[pl.ds(src, L)]
                gen16 = gen_v[pl.ds(src, L)]
                plsc.store_scatter(out_v, [rows, jnp.full((L,), D, jnp.int32)], age16)
                plsc.store_scatter(out_v, [rows, jnp.full((L,), D + 1, jnp.int32)], hr16)
                plsc.store_scatter(out_v, [rows, jnp.full((L,), D + 2, jnp.int32)], mo16)
                plsc.store_scatter(out_v, [rows, D + 3 + occ16], ones)
                plsc.store_scatter(out_v, [rows, 2 * D + 3 + gen16], ones)
                return 0
            lax.fori_loop(0, GROUPS, _grp, 0)

            pltpu.sync_copy(out_v, out_hbm.at[pl.ds(base + c * CH, CH)])

            if c != NCHUNK - 1:
                def _rst(g, _):
                    rows = g * L + iota
                    src = c * CH + g * L
                    occ16 = occ_v[pl.ds(src, L)]
                    gen16 = gen_v[pl.ds(src, L)]
                    plsc.store_scatter(out_v, [rows, D + 3 + occ16], zeros)
                    plsc.store_scatter(out_v, [rows, 2 * D + 3 + gen16], zeros)
                    return 0
                lax.fori_loop(0, GROUPS, _rst, 0)

    return sc_kernel


def kernel(user_id, age, hr_wk, month, occupation, gender, user_table):
    B = user_id.shape[0]
    V, D = user_table.shape
    sc = _make_sc_kernel(B, V, D)
    return sc(
        user_id.astype(jnp.int32),
        age.reshape(B),
        hr_wk.reshape(B),
        month.reshape(B),
        occupation.astype(jnp.int32),
        gender.astype(jnp.int32),
        user_table,
    )


# speedup vs baseline: 1.1942x; 1.1942x over previous
import functools

import jax
import jax.numpy as jnp
from jax import lax
from jax.experimental import pallas as pl
from jax.experimental.pallas import tpu as pltpu
from jax.experimental.pallas import tpu_sc as plsc

NC = 2
NS = 16
NW = NC * NS
L = 16


def _make_sc_kernel(B, V, D):
    OUT = 3 * D + 3
    OH0 = D + 3
    BPW = B // NW
    CH = 64
    NCHUNK = BPW // CH
    GROUPS = CH // L

    mesh = plsc.VectorSubcoreMesh(core_axis_name="c", subcore_axis_name="s")

    @functools.partial(
        pl.kernel,
        mesh=mesh,
        compiler_params=pltpu.CompilerParams(
            use_tc_tiling_on_sc=False, needs_layout_passes=False),
        out_type=jax.ShapeDtypeStruct((B, OUT), jnp.float32),
        scratch_types=[
            pltpu.VMEM((BPW,), jnp.int32),
            pltpu.VMEM((BPW,), jnp.float32),
            pltpu.VMEM((BPW,), jnp.float32),
            pltpu.VMEM((BPW,), jnp.float32),
            pltpu.VMEM((BPW,), jnp.int32),
            pltpu.VMEM((BPW,), jnp.int32),
            pltpu.VMEM((CH, D), jnp.float32),
            pltpu.VMEM((CH, D), jnp.float32),
            pltpu.VMEM((CH, OUT), jnp.float32),
            pltpu.VMEM((CH, OUT), jnp.float32),
            pltpu.SemaphoreType.DMA,
            pltpu.SemaphoreType.DMA,
            pltpu.SemaphoreType.DMA,
            pltpu.SemaphoreType.DMA,
        ],
    )
    def sc_kernel(uid_hbm, age_hbm, hr_hbm, mo_hbm, occ_hbm, gen_hbm,
                  table_hbm, out_hbm,
                  idx_v, age_v, hr_v, mo_v, occ_v, gen_v,
                  emb0, emb1, outb0, outb1, sg0, sg1, so0, so1):
        wid = lax.axis_index("s") * NC + lax.axis_index("c")
        base = wid * BPW
        emb = (emb0, emb1)
        outb = (outb0, outb1)
        sg = (sg0, sg1)
        so = (so0, so1)

        pltpu.sync_copy(uid_hbm.at[pl.ds(base, BPW)], idx_v)

        gathers = [None] * NCHUNK
        gathers[0] = pltpu.async_copy(
            table_hbm.at[idx_v.at[pl.ds(0, CH)]], emb[0], sg[0])

        pltpu.sync_copy(age_hbm.at[pl.ds(base, BPW)], age_v)
        pltpu.sync_copy(hr_hbm.at[pl.ds(base, BPW)], hr_v)
        pltpu.sync_copy(mo_hbm.at[pl.ds(base, BPW)], mo_v)
        pltpu.sync_copy(occ_hbm.at[pl.ds(base, BPW)], occ_v)
        pltpu.sync_copy(gen_hbm.at[pl.ds(base, BPW)], gen_v)

        iota = lax.iota(jnp.int32, L)
        zeros = jnp.zeros((L,), jnp.float32)
        ones = jnp.ones((L,), jnp.float32)

        def _zero(r, _):
            rsp = jnp.full((L,), r, jnp.int32)
            for g in range(16):
                plsc.store_scatter(outb0, [rsp, OH0 + g * L + iota], zeros)
                plsc.store_scatter(outb1, [rsp, OH0 + g * L + iota], zeros)
            return 0
        lax.fori_loop(0, CH, _zero, 0)

        copyouts = [None] * NCHUNK
        for c in range(NCHUNK):
            b = c % 2
            if c + 1 < NCHUNK:
                gathers[c + 1] = pltpu.async_copy(
                    table_hbm.at[idx_v.at[pl.ds((c + 1) * CH, CH)]],
                    emb[(c + 1) % 2], sg[(c + 1) % 2])
            if c >= 2:
                copyouts[c - 2].wait()
                def _rst(g, _, cc=c - 2):
                    rows = g * L + iota
                    src = cc * CH + g * L
                    plsc.store_scatter(
                        outb[b], [rows, OH0 + occ_v[pl.ds(src, L)]], zeros)
                    plsc.store_scatter(
                        outb[b], [rows, 2 * D + 3 + gen_v[pl.ds(src, L)]],
                        zeros)
                    return 0
                lax.fori_loop(0, GROUPS, _rst, 0)

            gathers[c].wait()

            def _row(r, _, b=b):
                for j in range(D // L):
                    plsc.store_scatter(
                        outb[b],
                        [jnp.full((L,), r, jnp.int32), j * L + iota],
                        emb[b][r, pl.ds(j * L, L)])
                return 0
            lax.fori_loop(0, CH, _row, 0)

            def _grp(g, _, c=c, b=b):
                rows = g * L + iota
                src = c * CH + g * L
                plsc.store_scatter(
                    outb[b], [rows, jnp.full((L,), D, jnp.int32)],
                    age_v[pl.ds(src, L)])
                plsc.store_scatter(
                    outb[b], [rows, jnp.full((L,), D + 1, jnp.int32)],
                    hr_v[pl.ds(src, L)])
                plsc.store_scatter(
                    outb[b], [rows, jnp.full((L,), D + 2, jnp.int32)],
                    mo_v[pl.ds(src, L)])
                plsc.store_scatter(
                    outb[b], [rows, OH0 + occ_v[pl.ds(src, L)]], ones)
                plsc.store_scatter(
                    outb[b], [rows, 2 * D + 3 + gen_v[pl.ds(src, L)]], ones)
                return 0
            lax.fori_loop(0, GROUPS, _grp, 0)

            copyouts[c] = pltpu.async_copy(
                outb[b], out_hbm.at[pl.ds(base + c * CH, CH)], so[b])

        copyouts[NCHUNK - 2].wait()
        copyouts[NCHUNK - 1].wait()

    return sc_kernel


def kernel(user_id, age, hr_wk, month, occupation, gender, user_table):
    B = user_id.shape[0]
    V, D = user_table.shape
    sc = _make_sc_kernel(B, V, D)
    return sc(
        user_id.astype(jnp.int32),
        age.reshape(B),
        hr_wk.reshape(B),
        month.reshape(B),
        occupation.astype(jnp.int32),
        gender.astype(jnp.int32),
        user_table,
    )


def try_build():
    B, D, V = 16384, 128, 100001
    return (
        kernel,
        (
            jax.ShapeDtypeStruct((B,), jnp.int32),
            jax.ShapeDtypeStruct((B, 1), jnp.float32),
            jax.ShapeDtypeStruct((B, 1), jnp.float32),
            jax.ShapeDtypeStruct((B, 1), jnp.float32),
            jax.ShapeDtypeStruct((B,), jnp.int32),
            jax.ShapeDtypeStruct((B,), jnp.int32),
            jax.ShapeDtypeStruct((V, D), jnp.float32),
        ),
    )


# speedup vs baseline: 1.4814x; 1.2405x over previous
import functools

import jax
import jax.numpy as jnp
from jax import lax
from jax.experimental import pallas as pl
from jax.experimental.pallas import tpu as pltpu
from jax.experimental.pallas import tpu_sc as plsc

NC = 2
NS = 16
NW = NC * NS
L = 16


def _make_sc_kernel(B, V, D):
    OUT = 3 * D + 3
    FP = 392
    FT = FP // 8
    CT = B // 128
    BPW = B // NW
    CTW = BPW // 128
    FTC = 7
    NCHUNK = FT // FTC
    GRP = BPW // L

    mesh = plsc.VectorSubcoreMesh(core_axis_name="c", subcore_axis_name="s")

    @functools.partial(
        pl.kernel,
        mesh=mesh,
        compiler_params=pltpu.CompilerParams(
            use_tc_tiling_on_sc=False, needs_layout_passes=False),
        out_type=jax.ShapeDtypeStruct((FT, CT, 8, 128), jnp.float32),
        scratch_types=[
            pltpu.VMEM((BPW,), jnp.int32),
            pltpu.VMEM((BPW,), jnp.float32),
            pltpu.VMEM((BPW,), jnp.float32),
            pltpu.VMEM((BPW,), jnp.float32),
            pltpu.VMEM((BPW,), jnp.int32),
            pltpu.VMEM((BPW,), jnp.int32),
            pltpu.VMEM((BPW, D), jnp.float32),
            pltpu.VMEM((FTC, CTW, 8, 128), jnp.float32),
            pltpu.SemaphoreType.DMA,
            pltpu.SemaphoreType.DMA,
            pltpu.SemaphoreType.DMA,
        ],
    )
    def sc_kernel(uid_hbm, age_hbm, hr_hbm, mo_hbm, occ_hbm, gen_hbm,
                  table_hbm, out_hbm,
                  idx_v, age_v, hr_v, mo_v, occ_v, gen_v, emb_v, buf,
                  sg, s_in, so):
        wid = lax.axis_index("s") * NC + lax.axis_index("c")
        base = wid * BPW
        ct0 = wid * CTW

        pltpu.sync_copy(uid_hbm.at[pl.ds(base, BPW)], idx_v)
        gather = pltpu.async_copy(table_hbm.at[idx_v], emb_v, sg)
        stage = [
            pltpu.async_copy(age_hbm.at[pl.ds(base, BPW)], age_v, s_in),
            pltpu.async_copy(hr_hbm.at[pl.ds(base, BPW)], hr_v, s_in),
            pltpu.async_copy(mo_hbm.at[pl.ds(base, BPW)], mo_v, s_in),
            pltpu.async_copy(occ_hbm.at[pl.ds(base, BPW)], occ_v, s_in),
            pltpu.async_copy(gen_hbm.at[pl.ds(base, BPW)], gen_v, s_in),
        ]

        iota = lax.iota(jnp.int32, L)
        iota_d = iota * D
        zeros = jnp.zeros((L,), jnp.float32)
        ones = jnp.ones((L,), jnp.float32)

        for cp in stage:
            cp.wait()
        gather.wait()

        copyout = None
        for c in range(NCHUNK):
            fb = c * FTC * 8
            fe = fb + FTC * 8
            n_emb = max(0, min(fe, D) - fb)
            n_oh = max(0, fe - max(fb, D + 3))

            if copyout is not None:
                copyout.wait()

            if n_emb:
                def _embf(i, _, fb=fb):
                    df = fb + (i >> 2)
                    ct = i & 3
                    ftl = (df - fb) >> 3
                    fr = df & 7
                    for k in range(8):
                        src = plsc.load_gather(
                            emb_v, [ct * 128 + k * L + iota,
                                    jnp.full((L,), df, jnp.int32)])
                        buf[ftl, ct, fr, pl.ds(k * L, L)] = src
                    return 0
                lax.fori_loop(0, n_emb * CTW, _embf, 0)

            if fb <= D and fe > D:
                ftl_s = (D - fb) >> 3
                def _scal(ct, _, ftl_s=ftl_s):
                    for k in range(8):
                        sl = pl.ds(ct * 128 + k * L, L)
                        buf[ftl_s, ct, 0, pl.ds(k * L, L)] = age_v[sl]
                        buf[ftl_s, ct, 1, pl.ds(k * L, L)] = hr_v[sl]
                        buf[ftl_s, ct, 2, pl.ds(k * L, L)] = mo_v[sl]
                    return 0
                lax.fori_loop(0, CTW, _scal, 0)

            if n_oh:
                f0 = max(fb, D + 3)
                def _zf(i, _, f0=f0, fb=fb):
                    df = f0 + (i >> 2)
                    ct = i & 3
                    ftl = (df - fb) >> 3
                    fr = df & 7
                    for k in range(8):
                        buf[ftl, ct, fr, pl.ds(k * L, L)] = zeros
                    return 0
                lax.fori_loop(0, n_oh * CTW, _zf, 0)

                def _ones(g, _, fb=fb, fe=fe):
                    col = (g & 7) * L + iota
                    ct = g >> 3
                    src = pl.ds(ct * 128 + (g & 7) * L, L)
                    fo = D + 3 + occ_v[src]
                    fg = 2 * D + 3 + gen_v[src]
                    ctv = jnp.full((L,), ct, jnp.int32)
                    plsc.store_scatter(
                        buf, [(fo - fb) >> 3, ctv, fo & 7, col], ones,
                        mask=(fo >= fb) & (fo < fe))
                    plsc.store_scatter(
                        buf, [(fg - fb) >> 3, ctv, fg & 7, col], ones,
                        mask=(fg >= fb) & (fg < fe))
                    return 0
                lax.fori_loop(0, GRP, _ones, 0)

            copyout = pltpu.async_copy(
                buf, out_hbm.at[pl.ds(c * FTC, FTC), pl.ds(ct0, CTW)], so)

        copyout.wait()

    return sc_kernel


def kernel(user_id, age, hr_wk, month, occupation, gender, user_table):
    B = user_id.shape[0]
    V, D = user_table.shape
    OUT = 3 * D + 3
    FP = OUT + (-OUT) % 8
    sc = _make_sc_kernel(B, V, D)
    t = sc(
        user_id.astype(jnp.int32),
        age.reshape(B),
        hr_wk.reshape(B),
        month.reshape(B),
        occupation.astype(jnp.int32),
        gender.astype(jnp.int32),
        user_table,
    )
    t = t.transpose(0, 2, 1, 3).reshape(FP, B).T
    return t[:, :OUT]


def try_build():
    B, D, V = 16384, 128, 100001
    return (
        kernel,
        (
            jax.ShapeDtypeStruct((B,), jnp.int32),
            jax.ShapeDtypeStruct((B, 1), jnp.float32),
            jax.ShapeDtypeStruct((B, 1), jnp.float32),
            jax.ShapeDtypeStruct((B, 1), jnp.float32),
            jax.ShapeDtypeStruct((B,), jnp.int32),
            jax.ShapeDtypeStruct((B,), jnp.int32),
            jax.ShapeDtypeStruct((V, D), jnp.float32),
        ),
    )


# speedup vs baseline: 2.1058x; 1.4216x over previous
import functools

import jax
import jax.numpy as jnp
from jax import lax
from jax.experimental import pallas as pl
from jax.experimental.pallas import tpu as pltpu
from jax.experimental.pallas import tpu_sc as plsc

NC = 2
NS = 16
NW = NC * NS
L = 16


def _make_sc_kernel(B, V, D):
    OUT = 3 * D + 3
    FP = 392
    FT = FP // 8
    CT = B // 128
    BPW = B // NW
    CTW = BPW // 128
    FTC = 7
    NCHUNK = FT // FTC
    GRP = BPW // L

    mesh = plsc.VectorSubcoreMesh(core_axis_name="c", subcore_axis_name="s")

    @functools.partial(
        pl.kernel,
        mesh=mesh,
        compiler_params=pltpu.CompilerParams(
            use_tc_tiling_on_sc=False, needs_layout_passes=False),
        out_type=jax.ShapeDtypeStruct((FT, CT, 8, 128), jnp.float32),
        scratch_types=[
            pltpu.VMEM((BPW,), jnp.int32),
            pltpu.VMEM((BPW,), jnp.float32),
            pltpu.VMEM((BPW,), jnp.float32),
            pltpu.VMEM((BPW,), jnp.float32),
            pltpu.VMEM((BPW,), jnp.int32),
            pltpu.VMEM((BPW,), jnp.int32),
            pltpu.VMEM((BPW, D), jnp.float32),
            pltpu.VMEM((FTC, CTW, 8, 128), jnp.float32),
            pltpu.VMEM((FTC, CTW, 8, 128), jnp.float32),
            pltpu.SemaphoreType.DMA,
            pltpu.SemaphoreType.DMA,
            pltpu.SemaphoreType.DMA,
            pltpu.SemaphoreType.DMA,
        ],
    )
    def sc_kernel(uid_hbm, age_hbm, hr_hbm, mo_hbm, occ_hbm, gen_hbm,
                  table_hbm, out_hbm,
                  idx_v, age_v, hr_v, mo_v, occ_v, gen_v, emb_v,
                  bufA, bufB, sg, s_in, soA, soB):
        wid = lax.axis_index("s") * NC + lax.axis_index("c")
        base = wid * BPW
        ct0 = wid * CTW
        bufs = (bufA, bufB)
        sos = (soA, soB)

        pltpu.sync_copy(uid_hbm.at[pl.ds(base, BPW)], idx_v)
        gather = pltpu.async_copy(table_hbm.at[idx_v], emb_v, sg)
        stage = [
            pltpu.async_copy(occ_hbm.at[pl.ds(base, BPW)], occ_v, s_in),
            pltpu.async_copy(gen_hbm.at[pl.ds(base, BPW)], gen_v, s_in),
            pltpu.async_copy(age_hbm.at[pl.ds(base, BPW)], age_v, s_in),
            pltpu.async_copy(hr_hbm.at[pl.ds(base, BPW)], hr_v, s_in),
            pltpu.async_copy(mo_hbm.at[pl.ds(base, BPW)], mo_v, s_in),
        ]

        iota = lax.iota(jnp.int32, L)
        zeros = jnp.zeros((L,), jnp.float32)
        ones = jnp.ones((L,), jnp.float32)

        def zero_fill(buf, ftl_lo):
            @plsc.parallel_loop(0, (FTC - ftl_lo) * CTW * 8, 1, unroll=2)
            def _zf(i):
                ftl = ftl_lo + (i >> 5)
                ct = (i >> 3) & 3
                fr = i & 7
                for k in range(8):
                    buf[ftl, ct, fr, pl.ds(k * L, L)] = zeros
        def scatter_vals(buf, c, vals):
            fb = c * FTC * 8
            fe = fb + FTC * 8

            def _ones(g, _):
                col = (g & 7) * L + iota
                ct = g >> 3
                src = pl.ds(ct * 128 + (g & 7) * L, L)
                fo = D + 3 + occ_v[src]
                fg = 2 * D + 3 + gen_v[src]
                ctv = jnp.full((L,), ct, jnp.int32)
                if fb < 2 * D + 3:
                    plsc.store_scatter(
                        buf, [(fo - fb) >> 3, ctv, fo & 7, col], vals,
                        mask=(fo >= fb) & (fo < fe))
                if fe > 2 * D + 3:
                    plsc.store_scatter(
                        buf, [(fg - fb) >> 3, ctv, fg & 7, col], vals,
                        mask=(fg >= fb) & (fg < fe))
                return 0
            lax.fori_loop(0, GRP, _ones, 0)

        def copy_out(buf, c, so):
            return pltpu.async_copy(
                buf, out_hbm.at[pl.ds(c * FTC, FTC), pl.ds(ct0, CTW)], so)

        stage[0].wait()
        stage[1].wait()
        last = [None, None]
        prevc = [None, None]
        for i, c in enumerate((3, 4, 5, 6)):
            b = i % 2
            if last[b] is not None:
                last[b].wait()
                scatter_vals(bufs[b], prevc[b], zeros)
            else:
                zero_fill(bufs[b], 0)
            scatter_vals(bufs[b], c, ones)
            prevc[b] = c
            last[b] = copy_out(bufs[b], c, sos[b])

        stage[2].wait()
        stage[3].wait()
        stage[4].wait()
        gather.wait()

        for i, c in enumerate((0, 1, 2)):
            b = i % 2
            last[b].wait()
            buf = bufs[b]
            fb = c * FTC * 8
            n_emb = min(fb + FTC * 8, D) - fb

            @plsc.parallel_loop(0, n_emb * CTW, 1, unroll=2)
            def _embf(i2, fb=fb, buf=buf):
                df = fb + (i2 >> 2)
                ct = i2 & 3
                ftl = (df - fb) >> 3
                fr = df & 7
                for k in range(8):
                    src = plsc.load_gather(
                        emb_v, [ct * 128 + k * L + iota,
                                jnp.full((L,), df, jnp.int32)])
                    buf[ftl, ct, fr, pl.ds(k * L, L)] = src

            if c == 2:
                def _scal(ct, _, buf=buf):
                    for k in range(8):
                        sl = pl.ds(ct * 128 + k * L, L)
                        buf[2, ct, 0, pl.ds(k * L, L)] = age_v[sl]
                        buf[2, ct, 1, pl.ds(k * L, L)] = hr_v[sl]
                        buf[2, ct, 2, pl.ds(k * L, L)] = mo_v[sl]
                    return 0
                lax.fori_loop(0, CTW, _scal, 0)
                zero_fill(buf, 3)

                def _z2(i2, _, buf=buf):
                    ct = i2 >> 3
                    k = i2 & 7
                    for fr in range(3, 8):
                        buf[2, ct, fr, pl.ds(k * L, L)] = zeros
                    return 0
                lax.fori_loop(0, CTW * 8, _z2, 0)
                scatter_vals(buf, c, ones)

            last[b] = copy_out(buf, c, sos[b])

        last[0].wait()
        last[1].wait()

    return sc_kernel


def kernel(user_id, age, hr_wk, month, occupation, gender, user_table):
    B = user_id.shape[0]
    V, D = user_table.shape
    OUT = 3 * D + 3
    FP = OUT + (-OUT) % 8
    sc = _make_sc_kernel(B, V, D)
    t = sc(
        user_id.astype(jnp.int32),
        age.reshape(B),
        hr_wk.reshape(B),
        month.reshape(B),
        occupation.astype(jnp.int32),
        gender.astype(jnp.int32),
        user_table,
    )
    t = t.transpose(0, 2, 1, 3).reshape(FP, B).T
    return t[:, :OUT]


def try_build():
    B, D, V = 16384, 128, 100001
    return (
        kernel,
        (
            jax.ShapeDtypeStruct((B,), jnp.int32),
            jax.ShapeDtypeStruct((B, 1), jnp.float32),
            jax.ShapeDtypeStruct((B, 1), jnp.float32),
            jax.ShapeDtypeStruct((B, 1), jnp.float32),
            jax.ShapeDtypeStruct((B,), jnp.int32),
            jax.ShapeDtypeStruct((B,), jnp.int32),
            jax.ShapeDtypeStruct((V, D), jnp.float32),
        ),
    )


# speedup vs baseline: 3.2257x; 1.5318x over previous
import functools

import jax
import jax.numpy as jnp
from jax import lax
from jax.experimental import pallas as pl
from jax.experimental.pallas import tpu as pltpu
from jax.experimental.pallas import tpu_sc as plsc

NC = 2
NS = 16
NW = NC * NS
L = 16


def _make_sc_kernel(B, V, D):
    OUT = 3 * D + 3
    FP = 392
    FT = FP // 8
    CT = B // 128
    BPW = B // NW
    CTW = BPW // 128
    FTC = 7
    NCHUNK = FT // FTC
    GRP = BPW // L

    mesh = plsc.VectorSubcoreMesh(core_axis_name="c", subcore_axis_name="s")

    @functools.partial(
        pl.kernel,
        mesh=mesh,
        compiler_params=pltpu.CompilerParams(
            use_tc_tiling_on_sc=False, needs_layout_passes=False),
        out_type=jax.ShapeDtypeStruct((FT, CT, 8, 128), jnp.float32),
        scratch_types=[
            pltpu.VMEM((BPW,), jnp.int32),
            pltpu.VMEM((BPW,), jnp.float32),
            pltpu.VMEM((BPW,), jnp.float32),
            pltpu.VMEM((BPW,), jnp.float32),
            pltpu.VMEM((BPW,), jnp.int32),
            pltpu.VMEM((BPW,), jnp.int32),
            pltpu.VMEM((BPW, D), jnp.float32),
            pltpu.VMEM((FTC, CTW, 8, 131), jnp.float32),
            pltpu.VMEM((FTC, CTW, 8, 131), jnp.float32),
            pltpu.SemaphoreType.DMA,
            pltpu.SemaphoreType.DMA,
            pltpu.SemaphoreType.DMA,
            pltpu.SemaphoreType.DMA,
        ],
    )
    def sc_kernel(uid_hbm, age_hbm, hr_hbm, mo_hbm, occ_hbm, gen_hbm,
                  table_hbm, out_hbm,
                  idx_v, age_v, hr_v, mo_v, occ_v, gen_v, emb_v,
                  bufA, bufB, sg, s_in, soA, soB):
        wid = lax.axis_index("s") * NC + lax.axis_index("c")
        base = wid * BPW
        ct0 = wid * CTW
        bufs = (bufA, bufB)
        sos = (soA, soB)

        pltpu.sync_copy(uid_hbm.at[pl.ds(base, BPW)], idx_v)
        gather = pltpu.async_copy(table_hbm.at[idx_v], emb_v, sg)
        stage = [
            pltpu.async_copy(occ_hbm.at[pl.ds(base, BPW)], occ_v, s_in),
            pltpu.async_copy(gen_hbm.at[pl.ds(base, BPW)], gen_v, s_in),
            pltpu.async_copy(age_hbm.at[pl.ds(base, BPW)], age_v, s_in),
            pltpu.async_copy(hr_hbm.at[pl.ds(base, BPW)], hr_v, s_in),
            pltpu.async_copy(mo_hbm.at[pl.ds(base, BPW)], mo_v, s_in),
        ]

        iota = lax.iota(jnp.int32, L)
        zeros = jnp.zeros((L,), jnp.float32)
        ones = jnp.ones((L,), jnp.float32)

        def zero_fill(buf, ftl_lo):
            @plsc.parallel_loop(0, (FTC - ftl_lo) * CTW * 8, 1, unroll=2)
            def _zf(i):
                ftl = ftl_lo + (i >> 5)
                ct = (i >> 3) & 3
                fr = i & 7
                for k in range(8):
                    buf[ftl, ct, fr, pl.ds(k * L, L)] = zeros
        def scatter_vals(buf, c, vals):
            fb = c * FTC * 8
            fe = fb + FTC * 8

            def _ones(g, _):
                col = (g & 7) * L + iota
                ct = g >> 3
                src = pl.ds(ct * 128 + (g & 7) * L, L)
                fo = D + 3 + occ_v[src]
                fg = 2 * D + 3 + gen_v[src]
                ctv = jnp.full((L,), ct, jnp.int32)
                if fb < 2 * D + 3:
                    plsc.store_scatter(
                        buf, [(fo - fb) >> 3, ctv, fo & 7, col], vals,
                        mask=(fo >= fb) & (fo < fe))
                if fe > 2 * D + 3:
                    plsc.store_scatter(
                        buf, [(fg - fb) >> 3, ctv, fg & 7, col], vals,
                        mask=(fg >= fb) & (fg < fe))
                return 0
            lax.fori_loop(0, GRP, _ones, 0)

        def copy_out(buf, c, so):
            return pltpu.async_copy(
                buf.at[:, :, :, pl.ds(0, 128)],
                out_hbm.at[pl.ds(c * FTC, FTC), pl.ds(ct0, CTW)], so)

        stage[0].wait()
        stage[1].wait()
        last = [None, None]
        prevc = [None, None]
        for i, c in enumerate((3, 4, 5, 6)):
            b = i % 2
            if last[b] is not None:
                last[b].wait()
                scatter_vals(bufs[b], prevc[b], zeros)
            else:
                zero_fill(bufs[b], 0)
            scatter_vals(bufs[b], c, ones)
            prevc[b] = c
            last[b] = copy_out(bufs[b], c, sos[b])

        stage[2].wait()
        stage[3].wait()
        stage[4].wait()
        gather.wait()

        for i, c in enumerate((0, 1, 2)):
            b = i % 2
            last[b].wait()
            buf = bufs[b]
            fb = c * FTC * 8
            n_emb = min(fb + FTC * 8, D) - fb

            f_hi = min(fb + FTC * 8, D)
            groups = []
            for f0 in range(0, D, L):
                if f0 + L > fb and f0 < f_hi:
                    fvec = f0 + iota
                    full = f0 >= fb and f0 + L <= f_hi
                    groups.append((
                        f0,
                        (fvec - fb) >> 3,
                        fvec & 7,
                        None if full else (fvec >= fb) & (fvec < f_hi),
                    ))

            @plsc.parallel_loop(0, BPW, 1, unroll=2)
            def _embf(bb, buf=buf, groups=groups):
                ctv = jnp.full((L,), bb >> 7, jnp.int32)
                colv = jnp.full((L,), bb & 127, jnp.int32)
                for f0, ftlv, frv, m in groups:
                    vals = emb_v[bb, pl.ds(f0, L)]
                    plsc.store_scatter(buf, [ftlv, ctv, frv, colv], vals,
                                       mask=m)

            if c == 2:
                def _scal(ct, _, buf=buf):
                    for k in range(8):
                        sl = pl.ds(ct * 128 + k * L, L)
                        buf[2, ct, 0, pl.ds(k * L, L)] = age_v[sl]
                        buf[2, ct, 1, pl.ds(k * L, L)] = hr_v[sl]
                        buf[2, ct, 2, pl.ds(k * L, L)] = mo_v[sl]
                    return 0
                lax.fori_loop(0, CTW, _scal, 0)
                zero_fill(buf, 3)

                def _z2(i2, _, buf=buf):
                    ct = i2 >> 3
                    k = i2 & 7
                    for fr in range(3, 8):
                        buf[2, ct, fr, pl.ds(k * L, L)] = zeros
                    return 0
                lax.fori_loop(0, CTW * 8, _z2, 0)
                scatter_vals(buf, c, ones)

            last[b] = copy_out(buf, c, sos[b])

        last[0].wait()
        last[1].wait()

    return sc_kernel


def kernel(user_id, age, hr_wk, month, occupation, gender, user_table):
    B = user_id.shape[0]
    V, D = user_table.shape
    OUT = 3 * D + 3
    FP = OUT + (-OUT) % 8
    sc = _make_sc_kernel(B, V, D)
    t = sc(
        user_id.astype(jnp.int32),
        age.reshape(B),
        hr_wk.reshape(B),
        month.reshape(B),
        occupation.astype(jnp.int32),
        gender.astype(jnp.int32),
        user_table,
    )
    t = t.transpose(0, 2, 1, 3).reshape(FP, B).T
    return t[:, :OUT]


def try_build():
    B, D, V = 16384, 128, 100001
    return (
        kernel,
        (
            jax.ShapeDtypeStruct((B,), jnp.int32),
            jax.ShapeDtypeStruct((B, 1), jnp.float32),
            jax.ShapeDtypeStruct((B, 1), jnp.float32),
            jax.ShapeDtypeStruct((B, 1), jnp.float32),
            jax.ShapeDtypeStruct((B,), jnp.int32),
            jax.ShapeDtypeStruct((B,), jnp.int32),
            jax.ShapeDtypeStruct((V, D), jnp.float32),
        ),
    )


# speedup vs baseline: 3.3154x; 1.0278x over previous
import functools

import jax
import jax.numpy as jnp
from jax import lax
from jax.experimental import pallas as pl
from jax.experimental.pallas import tpu as pltpu
from jax.experimental.pallas import tpu_sc as plsc

NC = 2
NS = 16
NW = NC * NS
L = 16


def _make_sc_kernel(B, V, D):
    OUT = 3 * D + 3
    FP = 392
    FT = FP // 8
    CT = B // 128
    BPW = B // NW
    CTW = BPW // 128
    FTC = 7
    NCHUNK = FT // FTC
    GRP = BPW // L

    mesh = plsc.VectorSubcoreMesh(core_axis_name="c", subcore_axis_name="s")

    @functools.partial(
        pl.kernel,
        mesh=mesh,
        compiler_params=pltpu.CompilerParams(
            use_tc_tiling_on_sc=False, needs_layout_passes=False,
            disable_bounds_checks=True, skip_device_barrier=True),
        out_type=jax.ShapeDtypeStruct((FT, CT, 8, 128), jnp.float32),
        scratch_types=[
            pltpu.VMEM((BPW,), jnp.int32),
            pltpu.VMEM((BPW,), jnp.float32),
            pltpu.VMEM((BPW,), jnp.float32),
            pltpu.VMEM((BPW,), jnp.float32),
            pltpu.VMEM((BPW,), jnp.int32),
            pltpu.VMEM((BPW,), jnp.int32),
            pltpu.VMEM((BPW, D), jnp.float32),
            pltpu.VMEM((FTC, CTW, 8, 131), jnp.float32),
            pltpu.VMEM((FTC, CTW, 8, 131), jnp.float32),
            pltpu.SemaphoreType.DMA,
            pltpu.SemaphoreType.DMA,
            pltpu.SemaphoreType.DMA,
            pltpu.SemaphoreType.DMA,
        ],
    )
    def sc_kernel(uid_hbm, age_hbm, hr_hbm, mo_hbm, occ_hbm, gen_hbm,
                  table_hbm, out_hbm,
                  idx_v, age_v, hr_v, mo_v, occ_v, gen_v, emb_v,
                  bufA, bufB, sg, s_in, soA, soB):
        wid = lax.axis_index("s") * NC + lax.axis_index("c")
        base = wid * BPW
        ct0 = wid * CTW
        bufs = (bufA, bufB)
        sos = (soA, soB)

        pltpu.sync_copy(uid_hbm.at[pl.ds(base, BPW)], idx_v)
        gather = pltpu.async_copy(table_hbm.at[idx_v], emb_v, sg)
        stage = [
            pltpu.async_copy(occ_hbm.at[pl.ds(base, BPW)], occ_v, s_in),
            pltpu.async_copy(gen_hbm.at[pl.ds(base, BPW)], gen_v, s_in),
            pltpu.async_copy(age_hbm.at[pl.ds(base, BPW)], age_v, s_in),
            pltpu.async_copy(hr_hbm.at[pl.ds(base, BPW)], hr_v, s_in),
            pltpu.async_copy(mo_hbm.at[pl.ds(base, BPW)], mo_v, s_in),
        ]

        iota = lax.iota(jnp.int32, L)
        zeros = jnp.zeros((L,), jnp.float32)
        ones = jnp.ones((L,), jnp.float32)

        def zero_fill(buf, ftl_lo):
            @plsc.parallel_loop(0, (FTC - ftl_lo) * CTW * 8, 1, unroll=2)
            def _zf(i):
                ftl = ftl_lo + (i >> 5)
                ct = (i >> 3) & 3
                fr = i & 7
                for k in range(8):
                    buf[ftl, ct, fr, pl.ds(k * L, L)] = zeros
        def scatter_vals(buf, c, vals):
            fb = c * FTC * 8
            fe = fb + FTC * 8

            def _ones(g, _):
                col = (g & 7) * L + iota
                ct = g >> 3
                src = pl.ds(ct * 128 + (g & 7) * L, L)
                fo = D + 3 + occ_v[src]
                fg = 2 * D + 3 + gen_v[src]
                ctv = jnp.full((L,), ct, jnp.int32)
                if fb < 2 * D + 3:
                    plsc.store_scatter(
                        buf, [(fo - fb) >> 3, ctv, fo & 7, col], vals,
                        mask=(fo >= fb) & (fo < fe))
                if fe > 2 * D + 3:
                    plsc.store_scatter(
                        buf, [(fg - fb) >> 3, ctv, fg & 7, col], vals,
                        mask=(fg >= fb) & (fg < fe))
                return 0
            lax.fori_loop(0, GRP, _ones, 0)

        def copy_out(buf, c, so):
            return pltpu.async_copy(
                buf.at[:, :, :, pl.ds(0, 128)],
                out_hbm.at[pl.ds(c * FTC, FTC), pl.ds(ct0, CTW)], so)

        stage[0].wait()
        stage[1].wait()
        last = [None, None]
        prevc = [None, None]
        for i, c in enumerate((3, 4, 5, 6)):
            b = i % 2
            if last[b] is not None:
                last[b].wait()
                scatter_vals(bufs[b], prevc[b], zeros)
            else:
                zero_fill(bufs[b], 0)
            scatter_vals(bufs[b], c, ones)
            prevc[b] = c
            last[b] = copy_out(bufs[b], c, sos[b])

        stage[2].wait()
        stage[3].wait()
        stage[4].wait()
        gather.wait()

        for i, c in enumerate((0, 1, 2)):
            b = i % 2
            last[b].wait()
            buf = bufs[b]
            fb = c * FTC * 8
            n_emb = min(fb + FTC * 8, D) - fb

            f_hi = min(fb + FTC * 8, D)
            groups = []
            for f0 in range(0, D, L):
                if f0 + L > fb and f0 < f_hi:
                    fvec = f0 + iota
                    full = f0 >= fb and f0 + L <= f_hi
                    groups.append((
                        f0,
                        (fvec - fb) >> 3,
                        fvec & 7,
                        None if full else (fvec >= fb) & (fvec < f_hi),
                    ))

            @plsc.parallel_loop(0, BPW, 1, unroll=4)
            def _embf(bb, buf=buf, groups=groups):
                ctv = jnp.full((L,), bb >> 7, jnp.int32)
                colv = jnp.full((L,), bb & 127, jnp.int32)
                for f0, ftlv, frv, m in groups:
                    vals = emb_v[bb, pl.ds(f0, L)]
                    plsc.store_scatter(buf, [ftlv, ctv, frv, colv], vals,
                                       mask=m)

            if c == 2:
                def _scal(ct, _, buf=buf):
                    for k in range(8):
                        sl = pl.ds(ct * 128 + k * L, L)
                        buf[2, ct, 0, pl.ds(k * L, L)] = age_v[sl]
                        buf[2, ct, 1, pl.ds(k * L, L)] = hr_v[sl]
                        buf[2, ct, 2, pl.ds(k * L, L)] = mo_v[sl]
                    return 0
                lax.fori_loop(0, CTW, _scal, 0)
                zero_fill(buf, 3)

                def _z2(i2, _, buf=buf):
                    ct = i2 >> 3
                    k = i2 & 7
                    for fr in range(3, 8):
                        buf[2, ct, fr, pl.ds(k * L, L)] = zeros
                    return 0
                lax.fori_loop(0, CTW * 8, _z2, 0)
                scatter_vals(buf, c, ones)

            last[b] = copy_out(buf, c, sos[b])

        last[0].wait()
        last[1].wait()

    return sc_kernel


def kernel(user_id, age, hr_wk, month, occupation, gender, user_table):
    B = user_id.shape[0]
    V, D = user_table.shape
    OUT = 3 * D + 3
    FP = OUT + (-OUT) % 8
    sc = _make_sc_kernel(B, V, D)
    t = sc(
        user_id.astype(jnp.int32),
        age.reshape(B),
        hr_wk.reshape(B),
        month.reshape(B),
        occupation.astype(jnp.int32),
        gender.astype(jnp.int32),
        user_table,
    )
    t = t.transpose(0, 2, 1, 3).reshape(FP, B).T
    return t[:, :OUT]


def try_build():
    B, D, V = 16384, 128, 100001
    return (
        kernel,
        (
            jax.ShapeDtypeStruct((B,), jnp.int32),
            jax.ShapeDtypeStruct((B, 1), jnp.float32),
            jax.ShapeDtypeStruct((B, 1), jnp.float32),
            jax.ShapeDtypeStruct((B, 1), jnp.float32),
            jax.ShapeDtypeStruct((B,), jnp.int32),
            jax.ShapeDtypeStruct((B,), jnp.int32),
            jax.ShapeDtypeStruct((V, D), jnp.float32),
        ),
    )
